# baseline (speedup 1.0000x reference)
import functools

import jax
import jax.numpy as jnp
from jax import lax
from jax.experimental import pallas as pl
from jax.experimental.pallas import tpu as pltpu
from jax.experimental.pallas import tpu_sc as plsc

BATCH = 32
TIME = 2048
FINAL_SIZE = 65536
STEP = FINAL_SIZE // TIME
L = 16

CHUNK = 2048
NCHUNK = FINAL_SIZE // CHUNK
VALS_PER_CHUNK = CHUNK // STEP
VREGS_PER_CHUNK = VALS_PER_CHUNK // L

_MESH = plsc.VectorSubcoreMesh(core_axis_name="c", subcore_axis_name="s")


@functools.partial(
    pl.kernel,
    out_type=jax.ShapeDtypeStruct((BATCH, 1, FINAL_SIZE), jnp.float32),
    mesh=_MESH,
    compiler_params=pltpu.CompilerParams(
        needs_layout_passes=False,
        disable_bounds_checks=True,
        skip_device_barrier=True,
    ),
    scratch_types=[
        pltpu.VMEM((TIME,), jnp.float32),
        pltpu.VMEM((CHUNK,), jnp.float32),
        pltpu.VMEM((CHUNK,), jnp.float32),
        pltpu.SemaphoreType.DMA,
        pltpu.SemaphoreType.DMA,
        pltpu.SemaphoreType.DMA,
    ],
)
def _impulse_sc(x_hbm, out_hbm, row_v, buf0, buf1, sem_in, sem0, sem1):
    wid = lax.axis_index("s") * 2 + lax.axis_index("c")

    in_h = pltpu.async_copy(x_hbm.at[wid], row_v, sem_in)

    zeros = jnp.zeros((L,), jnp.float32)

    @plsc.parallel_loop(0, CHUNK, L, unroll=8)
    def _(i):
        buf0[pl.ds(i, L)] = zeros
        buf1[pl.ds(i, L)] = zeros

    in_h.wait()

    mx16 = plsc.parallel_loop(
        0, TIME, L, unroll=4, carry=jnp.full((L,), -jnp.inf, jnp.float32)
    )(lambda i, acc: jnp.maximum(acc, row_v[pl.ds(i, L)]))
    mx = jnp.max(mx16)

    def exp_body(i, acc):
        e = jnp.exp(row_v[pl.ds(i, L)] - mx)
        row_v[pl.ds(i, L)] = e
        return acc + e

    s16 = plsc.parallel_loop(
        0, TIME, L, unroll=4, carry=jnp.zeros((L,), jnp.float32)
    )(exp_body)
    inv = jnp.full((L,), 1.0, jnp.float32) / (
        jnp.zeros((L,), jnp.float32) + jnp.sum(s16)
    )

    lane32 = lax.iota(jnp.int32, L) * STEP

    bufs = (buf0, buf1)
    sems = (sem0, sem1)

    def pair_body(p, _):
        for b in range(2):
            c = 2 * p + b
            buf = bufs[b]
            sem = sems[b]

            @pl.when(p > 0)
            def _():
                pltpu.make_async_copy(
                    buf, out_hbm.at[wid, 0, pl.ds((c - 2) * CHUNK, CHUNK)], sem
                ).wait()

            vbase = c * VALS_PER_CHUNK

            @plsc.parallel_loop(0, VALS_PER_CHUNK, L, unroll=2)
            def _(j):
                vals = row_v[pl.ds(vbase + j, L)] * inv
                plsc.store_scatter(buf, [lane32 + j * STEP], vals)

            pltpu.async_copy(
                buf, out_hbm.at[wid, 0, pl.ds(c * CHUNK, CHUNK)], sem
            )
        return 0

    lax.fori_loop(0, NCHUNK // 2, pair_body, 0)

    for b in range(2):
        c = NCHUNK - 2 + b
        pltpu.make_async_copy(
            bufs[b], out_hbm.at[wid, 0, pl.ds(c * CHUNK, CHUNK)], sems[b]
        ).wait()


def kernel(x):
    return _impulse_sc(x)


# speedup vs baseline: 1.0142x; 1.0142x over previous
import functools

import jax
import jax.numpy as jnp
from jax import lax
from jax.experimental import pallas as pl
from jax.experimental.pallas import tpu as pltpu
from jax.experimental.pallas import tpu_sc as plsc

BATCH = 32
TIME = 2048
FINAL_SIZE = 65536
STEP = FINAL_SIZE // TIME
L = 16

CHUNK = 8192
NCHUNK = FINAL_SIZE // CHUNK
VALS_PER_CHUNK = CHUNK // STEP
VREGS_PER_CHUNK = VALS_PER_CHUNK // L

_MESH = plsc.VectorSubcoreMesh(core_axis_name="c", subcore_axis_name="s")


@functools.partial(
    pl.kernel,
    out_type=jax.ShapeDtypeStruct((BATCH, 1, FINAL_SIZE), jnp.float32),
    mesh=_MESH,
    compiler_params=pltpu.CompilerParams(
        needs_layout_passes=False,
        disable_bounds_checks=True,
        skip_device_barrier=True,
    ),
    scratch_types=[
        pltpu.VMEM((TIME,), jnp.float32),
        pltpu.VMEM((CHUNK,), jnp.float32),
        pltpu.VMEM((CHUNK,), jnp.float32),
        pltpu.SemaphoreType.DMA,
        pltpu.SemaphoreType.DMA,
        pltpu.SemaphoreType.DMA,
    ],
)
def _impulse_sc(x_hbm, out_hbm, row_v, buf0, buf1, sem_in, sem0, sem1):
    wid = lax.axis_index("s") * 2 + lax.axis_index("c")

    in_h = pltpu.async_copy(x_hbm.at[wid], row_v, sem_in)

    zeros = jnp.zeros((L,), jnp.float32)

    @plsc.parallel_loop(0, CHUNK, L, unroll=8)
    def _(i):
        buf0[pl.ds(i, L)] = zeros
        buf1[pl.ds(i, L)] = zeros

    in_h.wait()

    mx16 = plsc.parallel_loop(
        0, TIME, L, unroll=4, carry=jnp.full((L,), -jnp.inf, jnp.float32)
    )(lambda i, acc: jnp.maximum(acc, row_v[pl.ds(i, L)]))
    mx = jnp.max(mx16)

    def exp_body(i, acc):
        e = jnp.exp(row_v[pl.ds(i, L)] - mx)
        row_v[pl.ds(i, L)] = e
        return acc + e

    s16 = plsc.parallel_loop(
        0, TIME, L, unroll=4, carry=jnp.zeros((L,), jnp.float32)
    )(exp_body)
    inv = jnp.full((L,), 1.0, jnp.float32) / (
        jnp.zeros((L,), jnp.float32) + jnp.sum(s16)
    )

    lane32 = lax.iota(jnp.int32, L) * STEP

    bufs = (buf0, buf1)
    sems = (sem0, sem1)

    def pair_body(p, _):
        for b in range(2):
            c = 2 * p + b
            buf = bufs[b]
            sem = sems[b]

            @pl.when(p > 0)
            def _():
                pltpu.make_async_copy(
                    buf, out_hbm.at[wid, 0, pl.ds((c - 2) * CHUNK, CHUNK)], sem
                ).wait()

            vbase = c * VALS_PER_CHUNK

            @plsc.parallel_loop(0, VALS_PER_CHUNK, L, unroll=2)
            def _(j):
                vals = row_v[pl.ds(vbase + j, L)] * inv
                plsc.store_scatter(buf, [lane32 + j * STEP], vals)

            pltpu.async_copy(
                buf, out_hbm.at[wid, 0, pl.ds(c * CHUNK, CHUNK)], sem
            )
        return 0

    lax.fori_loop(0, NCHUNK // 2, pair_body, 0)

    for b in range(2):
        c = NCHUNK - 2 + b
        pltpu.make_async_copy(
            bufs[b], out_hbm.at[wid, 0, pl.ds(c * CHUNK, CHUNK)], sems[b]
        ).wait()


def kernel(x):
    return _impulse_sc(x)
